# baseline (speedup 1.0000x reference)
import dataclasses
import functools

import jax
import jax.numpy as jnp
from jax import lax
from jax.experimental import pallas as pl
from jax.experimental.pallas import tpu as pltpu
from jax.experimental.pallas import tpu_sc as plsc

B = 16384
F = 128
NW = 32
B_PER_W = B // NW
UNIT = 128
UNITS_PER_W = B_PER_W // UNIT
LANES = 16
F_CHUNKS = F // LANES


def _mf_kernel(uidx_hbm, iidx_hbm, uemb_hbm, iemb_hbm, ubias_hbm, ibias_hbm,
               out_hbm, uidx_v, iidx_v, u_rows, i_rows, ub_rows, ib_rows,
               us_v, is_v, out_v):
    wid = lax.axis_index("s") * 2 + lax.axis_index("c")
    base_unit = wid * UNITS_PER_W

    pltpu.sync_copy(uidx_hbm.at[pl.ds(base_unit, UNITS_PER_W)], uidx_v)
    pltpu.sync_copy(iidx_hbm.at[pl.ds(base_unit, UNITS_PER_W)], iidx_v)

    lane = lax.iota(jnp.int32, LANES)
    zeros16 = jnp.zeros((LANES,), jnp.int32)

    for j in range(UNITS_PER_W):
        @pl.loop(0, UNIT, step=LANES)
        def _(k):
            us_v[pl.ds(k, LANES)] = jnp.right_shift(
                uidx_v[j, pl.ds(k, LANES)], 4)
            is_v[pl.ds(k, LANES)] = jnp.right_shift(
                iidx_v[j, pl.ds(k, LANES)], 4)

        pltpu.sync_copy(uemb_hbm.at[uidx_v.at[j]], u_rows)
        pltpu.sync_copy(iemb_hbm.at[iidx_v.at[j]], i_rows)
        pltpu.sync_copy(ubias_hbm.at[us_v], ub_rows)
        pltpu.sync_copy(ibias_hbm.at[is_v], ib_rows)

        @pl.loop(0, UNIT, step=LANES)
        def _(g):
            row_ids = g + lane
            ui = uidx_v[j, pl.ds(g, LANES)]
            ii = iidx_v[j, pl.ds(g, LANES)]
            bias = (plsc.load_gather(ub_rows, [row_ids, ui & 15])
                    + plsc.load_gather(ib_rows, [row_ids, ii & 15]))

            def fbody(fi, acc):
                fvec = fi * LANES + zeros16
                for t in range(LANES):
                    cols = fvec + t
                    acc = acc + (plsc.load_gather(u_rows, [row_ids, cols])
                                 * plsc.load_gather(i_rows, [row_ids, cols]))
                return acc

            out_v[pl.ds(g, LANES)] = lax.fori_loop(0, F // LANES, fbody, bias)

        pltpu.sync_copy(
            out_v, out_hbm.at[pl.ds(wid * B_PER_W + j * UNIT, UNIT)])


@jax.jit
def _mf(u_idx2d, i_idx2d, u_emb, i_emb, u_bias, i_bias):
    mesh = plsc.VectorSubcoreMesh(core_axis_name="c", subcore_axis_name="s")
    cp = pltpu.CompilerParams(needs_layout_passes=False,
                              use_tc_tiling_on_sc=False)
    run = pl.kernel(
        _mf_kernel,
        out_type=jax.ShapeDtypeStruct((B,), jnp.float32),
        mesh=mesh,
        compiler_params=cp,
        scratch_types=[
            pltpu.VMEM((UNITS_PER_W, UNIT), jnp.int32),
            pltpu.VMEM((UNITS_PER_W, UNIT), jnp.int32),
            pltpu.VMEM((UNIT, F), jnp.float32),
            pltpu.VMEM((UNIT, F), jnp.float32),
            pltpu.VMEM((UNIT, LANES), jnp.float32),
            pltpu.VMEM((UNIT, LANES), jnp.float32),
            pltpu.VMEM((UNIT,), jnp.int32),
            pltpu.VMEM((UNIT,), jnp.int32),
            pltpu.VMEM((UNIT,), jnp.float32),
        ],
    )
    return run(u_idx2d, i_idx2d, u_emb, i_emb, u_bias, i_bias)


def kernel(u_idx, i_idx, u_emb, i_emb, u_bias, i_bias):
    u_idx2d = u_idx.astype(jnp.int32).reshape(B // UNIT, UNIT)
    i_idx2d = i_idx.astype(jnp.int32).reshape(B // UNIT, UNIT)
    u_bias16 = u_bias.reshape(-1, LANES)
    i_bias16 = i_bias.reshape(-1, LANES)
    return _mf(u_idx2d, i_idx2d, u_emb, i_emb, u_bias16, i_bias16)


# speedup vs baseline: 1.1023x; 1.1023x over previous
import jax
import jax.numpy as jnp
from jax import lax
from jax.experimental import pallas as pl
from jax.experimental.pallas import tpu as pltpu
from jax.experimental.pallas import tpu_sc as plsc

B = 16384
F = 128
NW = 32
B_PER_W = B // NW
UNIT = 128
UNITS_PER_W = B_PER_W // UNIT
LANES = 16


def _mf_kernel(uidx_hbm, iidx_hbm, uemb_hbm, iemb_hbm, ubias_hbm, ibias_hbm,
               out_hbm, uidx_v, iidx_v, us_v, is_v,
               u_rows0, u_rows1, i_rows0, i_rows1,
               ub0, ub1, ib0, ib1, out_v, sem0, sem1):
    wid = lax.axis_index("s") * 2 + lax.axis_index("c")
    base_unit = wid * UNITS_PER_W

    pltpu.sync_copy(uidx_hbm.at[pl.ds(base_unit, UNITS_PER_W)], uidx_v)
    pltpu.sync_copy(iidx_hbm.at[pl.ds(base_unit, UNITS_PER_W)], iidx_v)

    lane = lax.iota(jnp.int32, LANES)
    zeros16 = jnp.zeros((LANES,), jnp.int32)

    for j in range(UNITS_PER_W):
        @pl.loop(0, UNIT, step=LANES)
        def _(k):
            us_v[j, pl.ds(k, LANES)] = jnp.right_shift(
                uidx_v[j, pl.ds(k, LANES)], 4)
            is_v[j, pl.ds(k, LANES)] = jnp.right_shift(
                iidx_v[j, pl.ds(k, LANES)], 4)

    ubuf = (u_rows0, u_rows1)
    ibuf = (i_rows0, i_rows1)
    ubb = (ub0, ub1)
    ibb = (ib0, ib1)
    sems = (sem0, sem1)
    handles = [None, None]

    def start_unit(j, p):
        handles[p] = (
            pltpu.async_copy(uemb_hbm.at[uidx_v.at[j]], ubuf[p], sems[p]),
            pltpu.async_copy(iemb_hbm.at[iidx_v.at[j]], ibuf[p], sems[p]),
            pltpu.async_copy(ubias_hbm.at[us_v.at[j]], ubb[p], sems[p]),
            pltpu.async_copy(ibias_hbm.at[is_v.at[j]], ibb[p], sems[p]),
        )

    start_unit(0, 0)
    for j in range(UNITS_PER_W):
        p = j & 1
        if j + 1 < UNITS_PER_W:
            start_unit(j + 1, 1 - p)
        for h in handles[p]:
            h.wait()
        u_rows, i_rows, ub_rows, ib_rows = ubuf[p], ibuf[p], ubb[p], ibb[p]

        @pl.loop(0, UNIT, step=LANES)
        def _(g):
            row_ids = g + lane
            ui = uidx_v[j, pl.ds(g, LANES)]
            ii = iidx_v[j, pl.ds(g, LANES)]
            acc = (plsc.load_gather(ub_rows, [row_ids, ui & 15])
                   + plsc.load_gather(ib_rows, [row_ids, ii & 15]))
            cols = zeros16
            for f in range(F):
                acc = acc + (plsc.load_gather(u_rows, [row_ids, cols])
                             * plsc.load_gather(i_rows, [row_ids, cols]))
                if f + 1 < F:
                    cols = cols + 1
            out_v[pl.ds(j * UNIT + g, LANES)] = acc

    pltpu.sync_copy(out_v, out_hbm.at[pl.ds(wid * B_PER_W, B_PER_W)])


@jax.jit
def _mf(u_idx2d, i_idx2d, u_emb, i_emb, u_bias, i_bias):
    mesh = plsc.VectorSubcoreMesh(core_axis_name="c", subcore_axis_name="s")
    cp = pltpu.CompilerParams(needs_layout_passes=False,
                              use_tc_tiling_on_sc=False)
    run = pl.kernel(
        _mf_kernel,
        out_type=jax.ShapeDtypeStruct((B,), jnp.float32),
        mesh=mesh,
        compiler_params=cp,
        scratch_types=[
            pltpu.VMEM((UNITS_PER_W, UNIT), jnp.int32),
            pltpu.VMEM((UNITS_PER_W, UNIT), jnp.int32),
            pltpu.VMEM((UNITS_PER_W, UNIT), jnp.int32),
            pltpu.VMEM((UNITS_PER_W, UNIT), jnp.int32),
            pltpu.VMEM((UNIT, F), jnp.float32),
            pltpu.VMEM((UNIT, F), jnp.float32),
            pltpu.VMEM((UNIT, F), jnp.float32),
            pltpu.VMEM((UNIT, F), jnp.float32),
            pltpu.VMEM((UNIT, LANES), jnp.float32),
            pltpu.VMEM((UNIT, LANES), jnp.float32),
            pltpu.VMEM((UNIT, LANES), jnp.float32),
            pltpu.VMEM((UNIT, LANES), jnp.float32),
            pltpu.VMEM((B_PER_W,), jnp.float32),
            pltpu.SemaphoreType.DMA,
            pltpu.SemaphoreType.DMA,
        ],
    )
    return run(u_idx2d, i_idx2d, u_emb, i_emb, u_bias, i_bias)


def kernel(u_idx, i_idx, u_emb, i_emb, u_bias, i_bias):
    u_idx2d = u_idx.astype(jnp.int32).reshape(B // UNIT, UNIT)
    i_idx2d = i_idx.astype(jnp.int32).reshape(B // UNIT, UNIT)
    u_bias16 = u_bias.reshape(-1, LANES)
    i_bias16 = i_bias.reshape(-1, LANES)
    return _mf(u_idx2d, i_idx2d, u_emb, i_emb, u_bias16, i_bias16)


# speedup vs baseline: 2.7159x; 2.4638x over previous
import jax
import jax.numpy as jnp
from jax import lax
from jax.experimental import pallas as pl
from jax.experimental.pallas import tpu as pltpu
from jax.experimental.pallas import tpu_sc as plsc

B = 16384
F = 128
NW = 32
B_PER_W = B // NW
UNIT = 128
UNITS_PER_W = B_PER_W // UNIT
LANES = 16


def _mf_kernel(uidx_hbm, iidx_hbm, uemb_hbm, iemb_hbm, ubias_hbm, ibias_hbm,
               out_hbm, uidx_v, iidx_v, us_v, is_v,
               u_rows0, u_rows1, i_rows0, i_rows1,
               ub0, ub1, ib0, ib1, part_v, out_v, sem0, sem1):
    wid = lax.axis_index("s") * 2 + lax.axis_index("c")
    base_unit = wid * UNITS_PER_W

    pltpu.sync_copy(uidx_hbm.at[pl.ds(base_unit, UNITS_PER_W)], uidx_v)
    pltpu.sync_copy(iidx_hbm.at[pl.ds(base_unit, UNITS_PER_W)], iidx_v)

    lane = lax.iota(jnp.int32, LANES)
    zeros16 = jnp.zeros((LANES,), jnp.int32)

    for j in range(UNITS_PER_W):
        @pl.loop(0, UNIT, step=LANES)
        def _(k):
            us_v[j, pl.ds(k, LANES)] = jnp.right_shift(
                uidx_v[j, pl.ds(k, LANES)], 4)
            is_v[j, pl.ds(k, LANES)] = jnp.right_shift(
                iidx_v[j, pl.ds(k, LANES)], 4)

    ubuf = (u_rows0, u_rows1)
    ibuf = (i_rows0, i_rows1)
    ubb = (ub0, ub1)
    ibb = (ib0, ib1)
    sems = (sem0, sem1)
    handles = [None, None]

    def start_unit(j, p):
        handles[p] = (
            pltpu.async_copy(uemb_hbm.at[uidx_v.at[j]], ubuf[p], sems[p]),
            pltpu.async_copy(iemb_hbm.at[iidx_v.at[j]], ibuf[p], sems[p]),
            pltpu.async_copy(ubias_hbm.at[us_v.at[j]], ubb[p], sems[p]),
            pltpu.async_copy(ibias_hbm.at[is_v.at[j]], ibb[p], sems[p]),
        )

    start_unit(0, 0)
    for j in range(UNITS_PER_W):
        p = j & 1
        if j + 1 < UNITS_PER_W:
            start_unit(j + 1, 1 - p)
        for h in handles[p]:
            h.wait()
        u_rows, i_rows, ub_rows, ib_rows = ubuf[p], ibuf[p], ubb[p], ibb[p]

        @pl.loop(0, UNIT, step=LANES)
        def _(g):
            row_ids = g + lane
            ui = uidx_v[j, pl.ds(g, LANES)]
            ii = iidx_v[j, pl.ds(g, LANES)]
            bias = (plsc.load_gather(ub_rows, [row_ids, ui & 15])
                    + plsc.load_gather(ib_rows, [row_ids, ii & 15]))
            for t in range(LANES):
                r = g + t
                prods = [u_rows[r, pl.ds(k * LANES, LANES)]
                         * i_rows[r, pl.ds(k * LANES, LANES)]
                         for k in range(F // LANES)]
                while len(prods) > 1:
                    prods = [prods[m] + prods[m + 1]
                             for m in range(0, len(prods), 2)]
                part_v[t, pl.ds(0, LANES)] = prods[0]
            accs = [bias, jnp.zeros((LANES,), jnp.float32),
                    jnp.zeros((LANES,), jnp.float32),
                    jnp.zeros((LANES,), jnp.float32)]
            for l in range(LANES):
                accs[l & 3] = accs[l & 3] + plsc.load_gather(
                    part_v, [lane, l + zeros16])
            total = (accs[0] + accs[1]) + (accs[2] + accs[3])
            out_v[pl.ds(j * UNIT + g, LANES)] = total

    pltpu.sync_copy(out_v, out_hbm.at[pl.ds(wid * B_PER_W, B_PER_W)])


@jax.jit
def _mf(u_idx2d, i_idx2d, u_emb, i_emb, u_bias, i_bias):
    mesh = plsc.VectorSubcoreMesh(core_axis_name="c", subcore_axis_name="s")
    cp = pltpu.CompilerParams(needs_layout_passes=False,
                              use_tc_tiling_on_sc=False)
    run = pl.kernel(
        _mf_kernel,
        out_type=jax.ShapeDtypeStruct((B,), jnp.float32),
        mesh=mesh,
        compiler_params=cp,
        scratch_types=[
            pltpu.VMEM((UNITS_PER_W, UNIT), jnp.int32),
            pltpu.VMEM((UNITS_PER_W, UNIT), jnp.int32),
            pltpu.VMEM((UNITS_PER_W, UNIT), jnp.int32),
            pltpu.VMEM((UNITS_PER_W, UNIT), jnp.int32),
            pltpu.VMEM((UNIT, F), jnp.float32),
            pltpu.VMEM((UNIT, F), jnp.float32),
            pltpu.VMEM((UNIT, F), jnp.float32),
            pltpu.VMEM((UNIT, F), jnp.float32),
            pltpu.VMEM((UNIT, LANES), jnp.float32),
            pltpu.VMEM((UNIT, LANES), jnp.float32),
            pltpu.VMEM((UNIT, LANES), jnp.float32),
            pltpu.VMEM((UNIT, LANES), jnp.float32),
            pltpu.VMEM((LANES, LANES + 1), jnp.float32),
            pltpu.VMEM((B_PER_W,), jnp.float32),
            pltpu.SemaphoreType.DMA,
            pltpu.SemaphoreType.DMA,
        ],
    )
    return run(u_idx2d, i_idx2d, u_emb, i_emb, u_bias, i_bias)


def kernel(u_idx, i_idx, u_emb, i_emb, u_bias, i_bias):
    u_idx2d = u_idx.astype(jnp.int32).reshape(B // UNIT, UNIT)
    i_idx2d = i_idx.astype(jnp.int32).reshape(B // UNIT, UNIT)
    u_bias16 = u_bias.reshape(-1, LANES)
    i_bias16 = i_bias.reshape(-1, LANES)
    return _mf(u_idx2d, i_idx2d, u_emb, i_emb, u_bias16, i_bias16)
